# baseline (speedup 1.0000x reference)
import functools

import jax
import jax.numpy as jnp
from jax import lax
from jax.experimental import pallas as pl
from jax.experimental.pallas import tpu as pltpu
from jax.experimental.pallas import tpu_sc as plsc

N = 10000
E = 320000
D = 128

NC = 2
NS = 16
NW = NC * NS

BATCH = 128
NROWS = E // BATCH
NPAD = 10240
ROWS_PER_TILE = NPAD // NS
NB = 80
EPW = NB * BATCH
EPAD = EPW * NW

A_HI = 24
A_CUT = A_HI * 80
A_TAIL = A_CUT + (NW - A_HI) * 72

_MESH = plsc.VectorSubcoreMesh(
    core_axis_name="c", subcore_axis_name="s", num_cores=NC, num_subcores=NS
)


@functools.partial(
    pl.kernel,
    out_type=jax.ShapeDtypeStruct((NC, NPAD), jnp.float32),
    mesh=_MESH,
    scratch_types=[
        pltpu.VMEM((80, BATCH), jnp.int32),
        pltpu.VMEM((8, BATCH), jnp.int32),
        pltpu.VMEM((BATCH,), jnp.float32),
        pltpu.VMEM((ROWS_PER_TILE,), jnp.float32),
        pltpu.VMEM_SHARED((NPAD,), jnp.float32),
        pltpu.SemaphoreType.DMA,
    ],
)
def _sc_deg(ei_hbm, lv_hbm, deg_out_hbm, idx_v, lv_v, ones_v, degbuf_v,
            deg_sh, sem):
    cid = lax.axis_index("c")
    sid = lax.axis_index("s")
    wid = cid * NS + sid
    base = jnp.where(wid < A_HI, 80 * wid, 72 * wid + A_CUT - 72 * A_HI)

    for j in range(ROWS_PER_TILE // 16):
        degbuf_v[pl.ds(16 * j, 16)] = jnp.zeros((16,), jnp.float32)
    pltpu.sync_copy(degbuf_v, deg_sh.at[pl.ds(sid * ROWS_PER_TILE, ROWS_PER_TILE)])
    for j in range(BATCH // 16):
        ones_v[pl.ds(16 * j, 16)] = jnp.ones((16,), jnp.float32)
    pltpu.sync_copy(ei_hbm.at[1, pl.ds(base, 72)], idx_v.at[pl.ds(0, 72)])

    @pl.when(wid < A_HI)
    def _():
        pltpu.sync_copy(ei_hbm.at[1, pl.ds(base + 72, 8)], idx_v.at[pl.ds(72, 8)])

    @pl.when(wid < 8)
    def _():
        pltpu.sync_copy(lv_hbm.at[1], lv_v)

    plsc.subcore_barrier()

    fired = [
        pltpu.async_copy(ones_v, deg_sh.at[idx_v.at[i]], sem, add=True)
        for i in range(72)
    ]
    for d in fired:
        d.wait()

    @pl.when(wid < A_HI)
    def _():
        fired2 = [
            pltpu.async_copy(ones_v, deg_sh.at[idx_v.at[72 + i]], sem, add=True)
            for i in range(8)
        ]
        for d in fired2:
            d.wait()

    @pl.when(wid < 8)
    def _():
        pltpu.async_copy(ones_v, deg_sh.at[lv_v.at[wid]], sem, add=True).wait()

    plsc.subcore_barrier()

    r0 = sid * ROWS_PER_TILE
    pltpu.sync_copy(deg_sh.at[pl.ds(r0, ROWS_PER_TILE)], degbuf_v)
    pltpu.sync_copy(degbuf_v, deg_out_hbm.at[cid, pl.ds(r0, ROWS_PER_TILE)])


WIN = 40
NWIN = NB // WIN


@functools.partial(
    pl.kernel,
    out_type=jax.ShapeDtypeStruct((NC, NPAD, D), jnp.float32),
    mesh=_MESH,
    scratch_types=[
        pltpu.VMEM((WIN, BATCH), jnp.int32),
        pltpu.VMEM((WIN, BATCH), jnp.int32),
        pltpu.VMEM((BATCH, D), jnp.float32),
        pltpu.VMEM((BATCH, D), jnp.float32),
        pltpu.VMEM_SHARED((NPAD, D), jnp.float32),
        pltpu.SemaphoreType.DMA,
        pltpu.SemaphoreType.DMA,
        pltpu.SemaphoreType.DMA,
        pltpu.SemaphoreType.DMA,
    ],
)
def _sc_msg(g_hbm, ei_hbm, acc_out_hbm,
            src_v, dst_v, buf_a, buf_b, acc_sh, sem_a, sem_b, sem_sa, sem_sb):
    cid = lax.axis_index("c")
    sid = lax.axis_index("s")
    wid = cid * NS + sid

    @pl.when(cid == 0)
    def _():
        for k in range(ROWS_PER_TILE // BATCH):
            r0 = sid * ROWS_PER_TILE + k * BATCH
            pltpu.sync_copy(g_hbm.at[pl.ds(r0, BATCH)], buf_a)
            pltpu.sync_copy(buf_a, acc_sh.at[pl.ds(r0, BATCH)])

    @pl.when(cid == 1)
    def _():
        pltpu.sync_copy(g_hbm.at[pl.ds(NPAD - BATCH, BATCH)], buf_a)
        for k in range(ROWS_PER_TILE // BATCH):
            r0 = sid * ROWS_PER_TILE + k * BATCH
            pltpu.sync_copy(buf_a, acc_sh.at[pl.ds(r0, BATCH)])

    plsc.subcore_barrier()

    def issue(i, buf, sem):
        pltpu.async_copy(g_hbm.at[src_v.at[i]], buf, sem)

    def drain(i, buf, sem):
        pltpu.make_async_copy(g_hbm.at[src_v.at[i]], buf, sem).wait()

    def scatter(i, buf, sem):
        pltpu.async_copy(buf, acc_sh.at[dst_v.at[i]], sem, add=True).wait()

    for w in range(NWIN):
        pltpu.sync_copy(ei_hbm.at[0, wid, pl.ds(w * WIN, WIN)], src_v)
        pltpu.sync_copy(ei_hbm.at[1, wid, pl.ds(w * WIN, WIN)], dst_v)
        issue(0, buf_a, sem_a)

        def body(j, carry):
            i = 2 * j
            issue(i + 1, buf_b, sem_b)
            drain(i, buf_a, sem_a)
            scatter(i, buf_a, sem_sa)
            issue(i + 2, buf_a, sem_a)
            drain(i + 1, buf_b, sem_b)
            scatter(i + 1, buf_b, sem_sb)
            return carry

        lax.fori_loop(0, WIN // 2 - 1, body, 0)
        i = WIN - 2
        issue(i + 1, buf_b, sem_b)
        drain(i, buf_a, sem_a)
        scatter(i, buf_a, sem_sa)
        drain(i + 1, buf_b, sem_b)
        scatter(i + 1, buf_b, sem_sb)
    plsc.subcore_barrier()

    for k in range(ROWS_PER_TILE // BATCH):
        r0 = sid * ROWS_PER_TILE + k * BATCH
        pltpu.sync_copy(acc_sh.at[pl.ds(r0, BATCH)], buf_a)
        pltpu.sync_copy(buf_a, acc_out_hbm.at[cid, pl.ds(r0, BATCH)])


_RB = 640


def _tc_g_body(x_ref, w_ref, deg_ref, g_ref):
    h = jnp.dot(x_ref[...], w_ref[...], preferred_element_type=jnp.float32)
    dis = lax.rsqrt(deg_ref[0, :] + deg_ref[1, :] + 1.0)
    g_ref[...] = h * dis[:, None]


def _tc_g(x_pad, w, deg2):
    return pl.pallas_call(
        _tc_g_body,
        grid=(NPAD // _RB,),
        in_specs=[
            pl.BlockSpec((_RB, D), lambda i: (i, 0)),
            pl.BlockSpec((D, D), lambda i: (0, 0)),
            pl.BlockSpec((NC, _RB), lambda i: (0, i)),
        ],
        out_specs=pl.BlockSpec((_RB, D), lambda i: (i, 0)),
        out_shape=jax.ShapeDtypeStruct((NPAD, D), jnp.float32),
    )(x_pad, w, deg2)


_OB = 2000


def _tc_out_body(a_ref, deg_ref, b_ref, o_ref):
    dis = lax.rsqrt(deg_ref[0, :, 0] + deg_ref[1, :, 0] + 1.0)
    s = a_ref[0] + a_ref[1]
    o_ref[...] = jnp.maximum(dis[:, None] * s + b_ref[...], 0.0)


def _tc_out(acc2, deg3, b2d):
    return pl.pallas_call(
        _tc_out_body,
        grid=(N // _OB,),
        in_specs=[
            pl.BlockSpec((NC, _OB, D), lambda i: (0, i, 0)),
            pl.BlockSpec((NC, _OB, 1), lambda i: (0, i, 0)),
            pl.BlockSpec((1, D), lambda i: (0, 0)),
        ],
        out_specs=pl.BlockSpec((_OB, D), lambda i: (i, 0)),
        out_shape=jax.ShapeDtypeStruct((N, D), jnp.float32),
    )(acc2, deg3, b2d)


def kernel(x, edge_index, W, b):
    ei32 = edge_index.astype(jnp.int32)
    ei3 = ei32.reshape(2, NROWS, BATCH)
    lv = jnp.concatenate(
        [ei3[:, A_TAIL:], jnp.full((2, 4, BATCH), N, jnp.int32)], axis=1
    )
    pad_idx = N + jnp.arange(EPAD - E, dtype=jnp.int32) % (NPAD - N)
    eip = jnp.concatenate(
        [ei32, jnp.broadcast_to(pad_idx, (2, EPAD - E))], axis=1
    ).reshape(2, NW, NB, BATCH)
    x_pad = jnp.zeros((NPAD, D), jnp.float32).at[:N].set(x)

    deg2 = _sc_deg(ei3, lv)
    g = _tc_g(x_pad, W, deg2)
    acc2 = _sc_msg(g, eip)
    return _tc_out(acc2, deg2.reshape(NC, NPAD, 1), b.reshape(1, D))


# speedup vs baseline: 1.0220x; 1.0220x over previous
import functools

import jax
import jax.numpy as jnp
from jax import lax
from jax.experimental import pallas as pl
from jax.experimental.pallas import tpu as pltpu
from jax.experimental.pallas import tpu_sc as plsc

N = 10000
E = 320000
D = 128

NC = 2
NS = 16
NW = NC * NS

BATCH = 128
NROWS = E // BATCH
NPAD = 10240
ROWS_PER_TILE = NPAD // NS
NB = 80
EPW = NB * BATCH
EPAD = EPW * NW

A_HI = 24
A_CUT = A_HI * 80
A_TAIL = A_CUT + (NW - A_HI) * 72

_MESH = plsc.VectorSubcoreMesh(
    core_axis_name="c", subcore_axis_name="s", num_cores=NC, num_subcores=NS
)


@functools.partial(
    pl.kernel,
    out_type=jax.ShapeDtypeStruct((NC, NPAD), jnp.float32),
    mesh=_MESH,
    scratch_types=[
        pltpu.VMEM((80, BATCH), jnp.int32),
        pltpu.VMEM((8, BATCH), jnp.int32),
        pltpu.VMEM((BATCH,), jnp.float32),
        pltpu.VMEM((ROWS_PER_TILE,), jnp.float32),
        pltpu.VMEM_SHARED((NPAD,), jnp.float32),
        pltpu.SemaphoreType.DMA,
    ],
)
def _sc_deg(ei_hbm, lv_hbm, deg_out_hbm, idx_v, lv_v, ones_v, degbuf_v,
            deg_sh, sem):
    cid = lax.axis_index("c")
    sid = lax.axis_index("s")
    wid = cid * NS + sid
    base = jnp.where(wid < A_HI, 80 * wid, 72 * wid + A_CUT - 72 * A_HI)

    for j in range(ROWS_PER_TILE // 16):
        degbuf_v[pl.ds(16 * j, 16)] = jnp.zeros((16,), jnp.float32)
    pltpu.sync_copy(degbuf_v, deg_sh.at[pl.ds(sid * ROWS_PER_TILE, ROWS_PER_TILE)])
    for j in range(BATCH // 16):
        ones_v[pl.ds(16 * j, 16)] = jnp.ones((16,), jnp.float32)
    pltpu.sync_copy(ei_hbm.at[1, pl.ds(base, 72)], idx_v.at[pl.ds(0, 72)])

    @pl.when(wid < A_HI)
    def _():
        pltpu.sync_copy(ei_hbm.at[1, pl.ds(base + 72, 8)], idx_v.at[pl.ds(72, 8)])

    @pl.when(wid < 8)
    def _():
        pltpu.sync_copy(lv_hbm.at[1], lv_v)

    plsc.subcore_barrier()

    fired = [
        pltpu.async_copy(ones_v, deg_sh.at[idx_v.at[i]], sem, add=True)
        for i in range(72)
    ]
    for d in fired:
        d.wait()

    @pl.when(wid < A_HI)
    def _():
        fired2 = [
            pltpu.async_copy(ones_v, deg_sh.at[idx_v.at[72 + i]], sem, add=True)
            for i in range(8)
        ]
        for d in fired2:
            d.wait()

    @pl.when(wid < 8)
    def _():
        pltpu.async_copy(ones_v, deg_sh.at[lv_v.at[wid]], sem, add=True).wait()

    plsc.subcore_barrier()

    r0 = sid * ROWS_PER_TILE
    pltpu.sync_copy(deg_sh.at[pl.ds(r0, ROWS_PER_TILE)], degbuf_v)
    pltpu.sync_copy(degbuf_v, deg_out_hbm.at[cid, pl.ds(r0, ROWS_PER_TILE)])


WIN = 40
NWIN = NB // WIN


@functools.partial(
    pl.kernel,
    out_type=jax.ShapeDtypeStruct((NC, NPAD, D), jnp.float32),
    mesh=_MESH,
    scratch_types=[
        pltpu.VMEM((WIN, BATCH), jnp.int32),
        pltpu.VMEM((WIN, BATCH), jnp.int32),
        pltpu.VMEM((BATCH, D), jnp.float32),
        pltpu.VMEM((BATCH, D), jnp.float32),
        pltpu.VMEM_SHARED((NPAD, D), jnp.float32),
        pltpu.SemaphoreType.DMA,
        pltpu.SemaphoreType.DMA,
        pltpu.SemaphoreType.DMA,
        pltpu.SemaphoreType.DMA,
    ],
)
def _sc_msg(g_hbm, ei_hbm, acc_out_hbm,
            src_v, dst_v, buf_a, buf_b, acc_sh, sem_a, sem_b, sem_sa, sem_sb):
    cid = lax.axis_index("c")
    sid = lax.axis_index("s")
    wid = cid * NS + sid

    @pl.when(cid == 0)
    def _():
        for k in range(ROWS_PER_TILE // BATCH):
            r0 = sid * ROWS_PER_TILE + k * BATCH
            pltpu.sync_copy(g_hbm.at[pl.ds(r0, BATCH)], buf_a)
            pltpu.sync_copy(buf_a, acc_sh.at[pl.ds(r0, BATCH)])

    @pl.when(cid == 1)
    def _():
        pltpu.sync_copy(g_hbm.at[pl.ds(NPAD - BATCH, BATCH)], buf_a)
        for k in range(ROWS_PER_TILE // BATCH):
            r0 = sid * ROWS_PER_TILE + k * BATCH
            pltpu.sync_copy(buf_a, acc_sh.at[pl.ds(r0, BATCH)])

    plsc.subcore_barrier()

    def issue(i, buf, sem):
        pltpu.async_copy(g_hbm.at[src_v.at[i]], buf, sem)

    def drain(i, buf, sem):
        pltpu.make_async_copy(g_hbm.at[src_v.at[i]], buf, sem).wait()

    def scatter(i, buf, sem):
        pltpu.async_copy(buf, acc_sh.at[dst_v.at[i]], sem, add=True).wait()

    for w in range(NWIN):
        pltpu.sync_copy(ei_hbm.at[0, wid, pl.ds(w * WIN, WIN)], src_v)
        pltpu.sync_copy(ei_hbm.at[1, wid, pl.ds(w * WIN, WIN)], dst_v)
        issue(0, buf_a, sem_a)

        def body(j, carry):
            i = 2 * j
            issue(i + 1, buf_b, sem_b)
            drain(i, buf_a, sem_a)
            scatter(i, buf_a, sem_sa)
            issue(i + 2, buf_a, sem_a)
            drain(i + 1, buf_b, sem_b)
            scatter(i + 1, buf_b, sem_sb)
            return carry

        lax.fori_loop(0, WIN // 2 - 1, body, 0)
        i = WIN - 2
        issue(i + 1, buf_b, sem_b)
        drain(i, buf_a, sem_a)
        scatter(i, buf_a, sem_sa)
        drain(i + 1, buf_b, sem_b)
        scatter(i + 1, buf_b, sem_sb)
    plsc.subcore_barrier()

    for k in range(ROWS_PER_TILE // BATCH):
        r0 = sid * ROWS_PER_TILE + k * BATCH
        pltpu.sync_copy(acc_sh.at[pl.ds(r0, BATCH)], buf_a)
        pltpu.sync_copy(buf_a, acc_out_hbm.at[cid, pl.ds(r0, BATCH)])


_RB = 1280


def _tc_g_body(x_ref, w_ref, deg_ref, g_ref):
    h = jnp.dot(x_ref[...], w_ref[...], preferred_element_type=jnp.float32)
    dis = lax.rsqrt(deg_ref[0, :] + deg_ref[1, :] + 1.0)
    g_ref[...] = h * dis[:, None]


def _tc_g(x_pad, w, deg2):
    return pl.pallas_call(
        _tc_g_body,
        grid=(NPAD // _RB,),
        in_specs=[
            pl.BlockSpec((_RB, D), lambda i: (i, 0)),
            pl.BlockSpec((D, D), lambda i: (0, 0)),
            pl.BlockSpec((NC, _RB), lambda i: (0, i)),
        ],
        out_specs=pl.BlockSpec((_RB, D), lambda i: (i, 0)),
        out_shape=jax.ShapeDtypeStruct((NPAD, D), jnp.float32),
    )(x_pad, w, deg2)


_OB = 2000


def _tc_out_body(a_ref, deg_ref, b_ref, o_ref):
    dis = lax.rsqrt(deg_ref[0, :, 0] + deg_ref[1, :, 0] + 1.0)
    s = a_ref[0] + a_ref[1]
    o_ref[...] = jnp.maximum(dis[:, None] * s + b_ref[...], 0.0)


def _tc_out(acc2, deg3, b2d):
    return pl.pallas_call(
        _tc_out_body,
        grid=(N // _OB,),
        in_specs=[
            pl.BlockSpec((NC, _OB, D), lambda i: (0, i, 0)),
            pl.BlockSpec((NC, _OB, 1), lambda i: (0, i, 0)),
            pl.BlockSpec((1, D), lambda i: (0, 0)),
        ],
        out_specs=pl.BlockSpec((_OB, D), lambda i: (i, 0)),
        out_shape=jax.ShapeDtypeStruct((N, D), jnp.float32),
    )(acc2, deg3, b2d)


def kernel(x, edge_index, W, b):
    ei32 = edge_index.astype(jnp.int32)
    ei3 = ei32.reshape(2, NROWS, BATCH)
    lv = jnp.concatenate(
        [ei3[:, A_TAIL:], jnp.full((2, 4, BATCH), N, jnp.int32)], axis=1
    )
    pad_idx = N + jnp.arange(EPAD - E, dtype=jnp.int32) % (NPAD - N)
    eip = jnp.concatenate(
        [ei32, jnp.broadcast_to(pad_idx, (2, EPAD - E))], axis=1
    ).reshape(2, NW, NB, BATCH)
    x_pad = jnp.zeros((NPAD, D), jnp.float32).at[:N].set(x)

    deg2 = _sc_deg(ei3, lv)
    g = _tc_g(x_pad, W, deg2)
    acc2 = _sc_msg(g, eip)
    return _tc_out(acc2, deg2.reshape(NC, NPAD, 1), b.reshape(1, D))
